# Initial kernel scaffold; baseline (speedup 1.0000x reference)
import jax
import jax.numpy as jnp
from jax.experimental import pallas as pl


def kernel(x_low, i_high, j_low, k, num_points_high, W, b):
    raise NotImplementedError("write your pallas kernel here")


# speedup vs baseline: 1.1434x; 1.1434x over previous
import jax
import jax.numpy as jnp
from jax import lax
from jax.experimental import pallas as pl
from jax.experimental.pallas import tpu as pltpu
from jax.experimental.pallas import tpu_sc as plsc

N_LOW = 50000
N_HIGH = 100000
E = 400000
C = 128
K = 27

NC, NS = 2, 16
NW = NC * NS

G1 = 512
CH1 = 25
E_PAD = NW * G1 * CH1

BE = 512
NB = E_PAD // BE

EC = E_PAD // NS
SB = 6400
NSB = EC // SB
NVS = SB // 16
RB = 10000
NPASS = 5
RPT = 632
RP = NS * RPT
TRASH = RP
TAIL = RB - 15 * RPT
BIAS_ROWS = 112
G3 = 128
PK_SHIFT = 8192

_MESH = dict(core_axis_name="c", subcore_axis_name="s", num_cores=NC,
             num_subcores=NS)
_NO_LAYOUT = pltpu.CompilerParams(needs_layout_passes=False)


def _gather_body(j_hbm, x_hbm, g_hbm, idx_v, rows_v, sem):
    cc = lax.axis_index("c")
    s = lax.axis_index("s")
    base = (s * NC + cc) * (G1 * CH1)

    def body(t, carry):
        off = base + t * G1
        pltpu.sync_copy(j_hbm.at[pl.ds(off, G1)], idx_v)
        pltpu.async_copy(x_hbm.at[idx_v], rows_v, sem).wait()
        pltpu.sync_copy(rows_v, g_hbm.at[pl.ds(off, G1)])
        return carry

    lax.fori_loop(0, CH1, body, 0)


def _matmul_body(k_ref, x_ref, w_ref, o_ref):
    kv = k_ref[...]
    kmin = jnp.min(kv)
    kmax = jnp.max(kv)
    x = x_ref[...]

    def body(kk, acc):
        xm = jnp.where(kv == kk, x, 0.0)
        wk = w_ref[kk]
        return acc + lax.dot_general(
            xm, wk, (((1,), (1,)), ((), ())),
            preferred_element_type=jnp.float32)

    o_ref[...] = lax.fori_loop(kmin, kmax + 1, body,
                               jnp.zeros((BE, C), jnp.float32))


def _scatter_body(i_hbm, ct_hbm, bias_hbm, out_hbm,
                  i_blk, lst, gid_st, rel_st, rows_v, bias_buf, spmem, sem):
    cc = lax.axis_index("c")
    s = lax.axis_index("s")

    pltpu.sync_copy(bias_hbm, bias_buf)

    lane = lax.iota(jnp.int32, 16)
    zeros16 = jnp.zeros((16,), jnp.int32)

    for p in range(NPASS):
        lo = (cc * NPASS + p) * RB

        for q in range(5):
            pltpu.sync_copy(bias_buf,
                            spmem.at[pl.ds(s * RPT + q * BIAS_ROWS,
                                           BIAS_ROWS)])
        pltpu.sync_copy(bias_buf.at[pl.ds(0, 72)],
                        spmem.at[pl.ds(s * RPT + 5 * BIAS_ROWS, 72)])
        plsc.subcore_barrier()

        for sb in range(NSB):
            pltpu.sync_copy(i_hbm.at[pl.ds(s * EC + sb * SB, SB)], i_blk)

            def scan_body(t, cur):
                iv = i_blk[pl.ds(t * 16, 16)]
                rel = iv - lo
                m = (rel >= 0) & (rel < RB)
                pk = rel * PK_SHIFT + (t * 16 + lane)
                mi = jnp.where(m, jnp.int32(1), jnp.int32(0))
                pos = cur + plsc.cumsum(mi) - 1
                plsc.store_scatter(lst, [pos], pk, mask=m)
                return cur + plsc.all_reduce_population_count(m)

            cur = lax.fori_loop(0, NVS, scan_body, zeros16)
            cnt = jnp.max(cur)
            nch = (cnt + G3 - 1) // G3
            gbase = s * EC + sb * SB

            def chunk_body(cidx, carry):
                for u in range(8):
                    pkv = lst[pl.ds(cidx * G3 + u * 16, 16)]
                    mv = (cidx * G3 + u * 16 + lane) < cnt
                    relv = jnp.where(mv, pkv // PK_SHIFT, TRASH)
                    gidv = jnp.where(mv, gbase + pkv % PK_SHIFT, 0)
                    gid_st[pl.ds(u * 16, 16)] = gidv
                    rel_st[0, pl.ds(u * 16, 16)] = relv
                pltpu.async_copy(ct_hbm.at[gid_st], rows_v, sem).wait()
                pltpu.sync_copy(rows_v, spmem.at[rel_st.at[0]], add=True)
                return carry

            lax.fori_loop(0, nch, chunk_body, 0)

        plsc.subcore_barrier()

        dst = lo + s * RPT

        @pl.when(s < NS - 1)
        def _():
            pltpu.sync_copy(spmem.at[pl.ds(s * RPT, RPT)],
                            out_hbm.at[pl.ds(dst, RPT)])

        @pl.when(s == NS - 1)
        def _():
            pltpu.sync_copy(spmem.at[pl.ds(s * RPT, TAIL)],
                            out_hbm.at[pl.ds(dst, TAIL)])

        plsc.subcore_barrier()


def kernel(x_low, i_high, j_low, k, num_points_high, W, b):
    pad = E_PAD - E
    i_dst = jnp.minimum(i_high, num_points_high - 1)
    ip = jnp.concatenate([i_dst, jnp.full((pad,), N_HIGH, jnp.int32)])
    jp = jnp.concatenate([j_low, jnp.zeros((pad,), jnp.int32)])
    kp = jnp.concatenate([k, jnp.full((pad,), K - 1, jnp.int32)])
    bias_blk = jnp.broadcast_to(b, (BIAS_ROWS, C))

    gathered = pl.kernel(
        _gather_body,
        out_type=jax.ShapeDtypeStruct((E_PAD, C), jnp.float32),
        mesh=plsc.VectorSubcoreMesh(**_MESH),
        scratch_types=[
            pltpu.VMEM((G1,), jnp.int32),
            pltpu.VMEM((G1, C), jnp.float32),
            pltpu.SemaphoreType.DMA,
        ],
        compiler_params=_NO_LAYOUT,
    )(jp, x_low)

    contrib = pl.pallas_call(
        _matmul_body,
        grid=(NB,),
        in_specs=[
            pl.BlockSpec((BE, 1), lambda i: (i, 0)),
            pl.BlockSpec((BE, C), lambda i: (i, 0)),
            pl.BlockSpec((K, C, C), lambda i: (0, 0, 0)),
        ],
        out_specs=pl.BlockSpec((BE, C), lambda i: (i, 0)),
        out_shape=jax.ShapeDtypeStruct((E_PAD, C), jnp.float32),
    )(kp.reshape(E_PAD, 1), gathered, W)

    out = pl.kernel(
        _scatter_body,
        out_type=jax.ShapeDtypeStruct((N_HIGH, C), jnp.float32),
        mesh=plsc.VectorSubcoreMesh(**_MESH),
        scratch_types=[
            pltpu.VMEM((SB,), jnp.int32),
            pltpu.VMEM((SB,), jnp.int32),
            pltpu.VMEM((G3,), jnp.int32),
            pltpu.VMEM((1, G3), jnp.int32),
            pltpu.VMEM((G3, C), jnp.float32),
            pltpu.VMEM((BIAS_ROWS, C), jnp.float32),
            pltpu.VMEM_SHARED((RP + 8, C), jnp.float32),
            pltpu.SemaphoreType.DMA,
        ],
        compiler_params=_NO_LAYOUT,
    )(ip, contrib, bias_blk)

    return out


assert 15 * RPT + TAIL == RB and NC * NPASS * RB == N_HIGH
assert TAIL % 8 == 0 and RB % 8 == 0 and RPT % 8 == 0
assert RPT == 5 * BIAS_ROWS + 72
